# baseline (speedup 1.0000x reference)
import functools

import jax
import jax.numpy as jnp
from jax import lax
from jax.experimental import pallas as pl
from jax.experimental.pallas import tpu as pltpu
from jax.experimental.pallas import tpu_sc as plsc

_NC = 2
_NS = 16
_NW = _NC * _NS

_CH = 16
_NBUF = 2


def _sc_span_gather(feat, wemb2, idxp, *, S, H, WD, B, n_total):
    OUT_D = 2 * H + WD
    spw = n_total // _NW
    nchunk = spw // _CH
    wpb = (n_total // B) // spw
    mesh = plsc.VectorSubcoreMesh(core_axis_name="c", subcore_axis_name="s")

    N = n_total // B

    @functools.partial(
        pl.kernel,
        out_type=jax.ShapeDtypeStruct((B, N, OUT_D), jnp.float32),
        mesh=mesh,
        compiler_params=pltpu.CompilerParams(use_tc_tiling_on_sc=True),
        scratch_types=[
            pltpu.VMEM((3, spw), jnp.int32),
            [pltpu.VMEM((_CH,), jnp.int32) for _ in range(_NBUF)],
            [pltpu.VMEM((_CH, OUT_D), jnp.float32) for _ in range(_NBUF)],
            [pltpu.VMEM((_CH, 2 * WD), jnp.float32) for _ in range(_NBUF)],
            [pltpu.SemaphoreType.DMA for _ in range(_NBUF)],
            [pltpu.SemaphoreType.DMA for _ in range(_NBUF)],
        ],
    )
    def k(feat_hbm, wemb_hbm, idx_hbm, out_hbm, idxw, widx, obuf, wbuf,
          semG, semS):
        wid = lax.axis_index("s") * _NC + lax.axis_index("c")
        base = wid * spw
        bi = wid // wpb
        lbase = (wid % wpb) * spw
        b_s = bi * S

        pltpu.sync_copy(idx_hbm.at[:, pl.ds(base, spw)], idxw)

        def scat_desc(c, b):
            return pltpu.make_async_copy(
                obuf[b], out_hbm.at[bi, pl.ds(lbase + c * _CH, _CH)], semS[b])

        def gath_descs(c, b):
            co = c * _CH
            return (
                pltpu.make_async_copy(
                    feat_hbm.at[idxw.at[0, pl.ds(co, _CH)]],
                    obuf[b].at[:, pl.ds(0, H)], semG[b]),
                pltpu.make_async_copy(
                    feat_hbm.at[idxw.at[1, pl.ds(co, _CH)]],
                    obuf[b].at[:, pl.ds(H, H)], semG[b]),
                pltpu.make_async_copy(wemb_hbm.at[widx[b]], wbuf[b], semG[b]),
            )

        def issue_gathers(c, b):
            wvec = idxw[2, pl.ds(c * _CH, _CH)]
            widx[b][...] = lax.shift_right_logical(wvec, 1)
            for d in gath_descs(c, b):
                d.start()

        issue_gathers(0, 0)

        def sub(c, b):
            for d in gath_descs(c, b):
                d.wait()

            @pl.when(c + 1 < nchunk)
            def _(b=b):
                @pl.when(c >= 1)
                def _(b=b):
                    scat_desc(c + 1, b ^ 1).wait()
                issue_gathers(c + 1, b ^ 1)

            co = c * _CH
            wvec = idxw[2, pl.ds(co, _CH)]
            evec = idxw[1, pl.ds(co, _CH)] - b_s
            z = jnp.zeros((16,), jnp.float32)
            for j in range(_CH):
                off = (wvec[j] & 1) * WD
                for kk in range(WD // 16):
                    obuf[b][j, pl.ds(2 * H + kk * 16, 16)] = (
                        wbuf[b][j, pl.ds(off + kk * 16, 16)])

                @pl.when(evec[j] <= 0)
                def _zero(j=j, b=b):
                    def zrow(kk, _):
                        obuf[b][j, pl.ds(kk * 16, 16)] = z
                        return 0
                    lax.fori_loop(0, OUT_D // 16, zrow, 0)

            scat_desc(c, b).start()

        def body(t, _):
            for b in range(_NBUF):
                sub(t * _NBUF + b, b)
            return 0

        lax.fori_loop(0, nchunk // _NBUF, body, 0)
        for b in range(_NBUF):
            scat_desc(0, b).wait()

    return k(feat, wemb2, idxp)


def kernel(features, clause_candidates, width_embedding):
    B, S, H = features.shape
    N = clause_candidates.shape[1]
    WD = width_embedding.shape[1]

    cc = clause_candidates.astype(jnp.int32)
    starts = cc[:, :, 0]
    ends = cc[:, :, 1]
    boff = (jnp.arange(B, dtype=jnp.int32) * S)[:, None]
    idxp = jnp.stack([
        (starts + boff).reshape(B * N),
        (ends + boff).reshape(B * N),
        (ends - starts).reshape(B * N),
    ])

    return _sc_span_gather(
        features.reshape(B * S, H),
        width_embedding.reshape(-1, 2 * WD),
        idxp, S=S, H=H, WD=WD, B=B, n_total=B * N,
    )


# speedup vs baseline: 1.0587x; 1.0587x over previous
import functools

import jax
import jax.numpy as jnp
from jax import lax
from jax.experimental import pallas as pl
from jax.experimental.pallas import tpu as pltpu
from jax.experimental.pallas import tpu_sc as plsc

_NC = 2
_NS = 16
_NW = _NC * _NS

_CH = 16
_NBUF = 2


def _sc_span_gather(feat, wemb2, idxp, *, S, H, WD, B, n_total):
    OUT_D = 2 * H + WD
    spw = n_total // _NW
    nchunk = spw // _CH
    wpb = (n_total // B) // spw
    mesh = plsc.VectorSubcoreMesh(core_axis_name="c", subcore_axis_name="s")

    @functools.partial(
        pl.kernel,
        out_type=jax.ShapeDtypeStruct((n_total, OUT_D), jnp.float32),
        mesh=mesh,
        compiler_params=pltpu.CompilerParams(use_tc_tiling_on_sc=True),
        scratch_types=[
            pltpu.VMEM((3, spw), jnp.int32),
            [pltpu.VMEM((_CH,), jnp.int32) for _ in range(_NBUF)],
            [pltpu.VMEM((_CH, OUT_D), jnp.float32) for _ in range(_NBUF)],
            [pltpu.VMEM((_CH, 2 * WD), jnp.float32) for _ in range(_NBUF)],
            [pltpu.SemaphoreType.DMA for _ in range(_NBUF)],
            [pltpu.SemaphoreType.DMA for _ in range(_NBUF)],
        ],
    )
    def k(feat_hbm, wemb_hbm, idx_hbm, out_hbm, idxw, widx, obuf, wbuf,
          semG, semS):
        wid = lax.axis_index("s") * _NC + lax.axis_index("c")
        base = wid * spw
        b_s = (wid // wpb) * S

        pltpu.sync_copy(idx_hbm.at[:, pl.ds(base, spw)], idxw)

        def scat_desc(c, b):
            return pltpu.make_async_copy(
                obuf[b], out_hbm.at[pl.ds(base + c * _CH, _CH)], semS[b])

        def gath_descs(c, b):
            co = c * _CH
            return (
                pltpu.make_async_copy(
                    feat_hbm.at[idxw.at[0, pl.ds(co, _CH)]],
                    obuf[b].at[:, pl.ds(0, H)], semG[b]),
                pltpu.make_async_copy(
                    feat_hbm.at[idxw.at[1, pl.ds(co, _CH)]],
                    obuf[b].at[:, pl.ds(H, H)], semG[b]),
                pltpu.make_async_copy(wemb_hbm.at[widx[b]], wbuf[b], semG[b]),
            )

        def issue_gathers(c, b):
            wvec = idxw[2, pl.ds(c * _CH, _CH)]
            widx[b][...] = lax.shift_right_logical(wvec, 1)
            for d in gath_descs(c, b):
                d.start()

        issue_gathers(0, 0)

        def sub(c, b):
            for d in gath_descs(c, b):
                d.wait()

            @pl.when(c + 1 < nchunk)
            def _(b=b):
                @pl.when(c >= 1)
                def _(b=b):
                    scat_desc(c + 1, b ^ 1).wait()
                issue_gathers(c + 1, b ^ 1)

            co = c * _CH
            wvec = idxw[2, pl.ds(co, _CH)]
            evec = idxw[1, pl.ds(co, _CH)] - b_s
            z = jnp.zeros((16,), jnp.float32)
            for j in range(_CH):
                off = (wvec[j] & 1) * WD
                for kk in range(WD // 16):
                    obuf[b][j, pl.ds(2 * H + kk * 16, 16)] = (
                        wbuf[b][j, pl.ds(off + kk * 16, 16)])

                @pl.when(evec[j] <= 0)
                def _zero(j=j, b=b):
                    def zrow(kk, _):
                        obuf[b][j, pl.ds(kk * 16, 16)] = z
                        return 0
                    lax.fori_loop(0, OUT_D // 16, zrow, 0)

            scat_desc(c, b).start()

        def body(t, _):
            for b in range(_NBUF):
                sub(t * _NBUF + b, b)
            return 0

        lax.fori_loop(0, nchunk // _NBUF, body, 0)
        for b in range(_NBUF):
            scat_desc(0, b).wait()

    return k(feat, wemb2, idxp)


def kernel(features, clause_candidates, width_embedding):
    B, S, H = features.shape
    N = clause_candidates.shape[1]
    WD = width_embedding.shape[1]

    cc = clause_candidates.astype(jnp.int32)
    starts = cc[:, :, 0]
    ends = cc[:, :, 1]
    boff = (jnp.arange(B, dtype=jnp.int32) * S)[:, None]
    idxp = jnp.stack([
        (starts + boff).reshape(B * N),
        (ends + boff).reshape(B * N),
        (ends - starts).reshape(B * N),
    ])

    out = _sc_span_gather(
        features.reshape(B * S, H),
        width_embedding.reshape(-1, 2 * WD),
        idxp, S=S, H=H, WD=WD, B=B, n_total=B * N,
    )
    return out.reshape(B, N, 2 * H + WD)
